# baseline (speedup 1.0000x reference)
import functools

import jax
import jax.numpy as jnp
import numpy as np
from jax import lax
from jax.experimental import pallas as pl
from jax.experimental.pallas import tpu as pltpu
from jax.experimental.pallas import tpu_sc as plsc

BATCH = 16384
NUM_FEATURES = 26
FEATURE_DIM = 28
OUT_DIM = 32

NUM_WORKERS = 32
ROWS_PER_WORKER = BATCH // NUM_WORKERS
CHUNK_ROWS = 64
NUM_CHUNKS = ROWS_PER_WORKER // CHUNK_ROWS
CFLAT = CHUNK_ROWS * NUM_FEATURES
GATHER_SLICE = 128
NUM_GATHERS = CFLAT // GATHER_SLICE
IDX_ROWS = CFLAT // 128

_FOFF = ((np.arange(CFLAT, dtype=np.int64) % NUM_FEATURES) * 100000 + 1).astype(
    np.int32
).reshape(IDX_ROWS, 128)


def _sc_body(feat_hbm, codes_hbm, tmpl_hbm, foff_hbm, out_hbm,
             codes_v, idx_v, foff_v, rows_v, out_tile, tmpl_v, sem):
    wid = lax.axis_index("s") * 2 + lax.axis_index("c")
    base_row = wid * (ROWS_PER_WORKER * NUM_FEATURES // 128)

    pltpu.sync_copy(foff_hbm, foff_v)
    pltpu.sync_copy(tmpl_hbm, tmpl_v)

    lane = lax.iota(jnp.int32, 16)
    head_mask = lane < 4
    cidx0 = jnp.maximum(lane - 4, 0)
    cidx1 = lane + 12

    for ch in range(NUM_CHUNKS):
        roff = base_row + ch * IDX_ROWS
        pltpu.sync_copy(codes_hbm.at[pl.ds(roff, IDX_ROWS)], codes_v)

        def idx_body(q, _):
            for k in range(8):
                s = k * 16
                c = codes_v[q, pl.ds(s, 16)]
                idx_v[q, pl.ds(s, 16)] = c.astype(jnp.int32) + foff_v[q, pl.ds(s, 16)]
            return 0

        lax.fori_loop(0, IDX_ROWS, idx_body, 0)

        gathers = [
            pltpu.async_copy(
                feat_hbm.at[idx_v.at[j]],
                rows_v.at[pl.ds(j * GATHER_SLICE, GATHER_SLICE)],
                sem,
            )
            for j in range(NUM_GATHERS)
        ]
        for g in gathers:
            g.wait()

        def repack_body(r, _):
            f = lax.rem(r, NUM_FEATURES)
            t = tmpl_v[f, :]
            rsplat = jnp.full((16,), r, jnp.int32)
            g0 = plsc.load_gather(rows_v, [rsplat, cidx0])
            ve = jnp.where(head_mask, t, g0)
            vo = plsc.load_gather(rows_v, [rsplat, cidx1])
            out_tile[r, pl.ds(0, 16)] = ve
            out_tile[r, pl.ds(16, 16)] = vo
            return 0

        lax.fori_loop(0, CFLAT, repack_body, 0)

        pltpu.sync_copy(out_tile, out_hbm.at[pl.ds(roff * 128, CFLAT)])


@jax.jit
def _run(feature_embedding, codes_2d, tmpl, foff):
    mesh = plsc.VectorSubcoreMesh(core_axis_name="c", subcore_axis_name="s")
    k = functools.partial(
        pl.kernel,
        mesh=mesh,
        out_type=jax.ShapeDtypeStruct((BATCH * NUM_FEATURES, OUT_DIM), jnp.float32),
        scratch_types=[
            pltpu.VMEM((IDX_ROWS, 128), jnp.float32),
            pltpu.VMEM((IDX_ROWS, 128), jnp.int32),
            pltpu.VMEM((IDX_ROWS, 128), jnp.int32),
            pltpu.VMEM((CFLAT, OUT_DIM), jnp.float32),
            pltpu.VMEM((CFLAT, OUT_DIM), jnp.float32),
            pltpu.VMEM((NUM_FEATURES, 16), jnp.float32),
            pltpu.SemaphoreType.DMA,
        ],
        compiler_params=pltpu.CompilerParams(use_tc_tiling_on_sc=False,
                                             needs_layout_passes=False),
    )(_sc_body)
    return k(feature_embedding, codes_2d, tmpl, foff)


def kernel(inputs, feature_embedding, shared_embedding):
    feature_embedding = jnp.pad(feature_embedding, ((0, 0), (0, OUT_DIM - FEATURE_DIM)))
    codes_2d = inputs.reshape(BATCH * NUM_FEATURES // 128, 128)
    tmpl = jnp.pad(shared_embedding, ((0, 0), (0, 12)))
    out = _run(feature_embedding, codes_2d, tmpl, jnp.asarray(_FOFF))
    return out.reshape(BATCH, NUM_FEATURES, OUT_DIM)


# speedup vs baseline: 1.0012x; 1.0012x over previous
import functools

import jax
import jax.numpy as jnp
import numpy as np
from jax import lax
from jax.experimental import pallas as pl
from jax.experimental.pallas import tpu as pltpu
from jax.experimental.pallas import tpu_sc as plsc

BATCH = 16384
NUM_FEATURES = 26
FEATURE_DIM = 28
OUT_DIM = 32

NUM_WORKERS = 32
ROWS_PER_WORKER = BATCH // NUM_WORKERS
CHUNK_ROWS = 64
NUM_CHUNKS = ROWS_PER_WORKER // CHUNK_ROWS
CFLAT = CHUNK_ROWS * NUM_FEATURES
GATHER_SLICE = 128
NUM_GATHERS = CFLAT // GATHER_SLICE
IDX_ROWS = CFLAT // 128

_FOFF = ((np.arange(CFLAT, dtype=np.int64) % NUM_FEATURES) * 100000 + 1).astype(
    np.int32
).reshape(IDX_ROWS, 128)


def _sc_body(feat_hbm, codes_hbm, tmpl_hbm, foff_hbm, out_hbm,
             codes_v, idx_v, foff_v, rows_v, out_tile, tmpl_v, sem):
    wid = lax.axis_index("s") * 2 + lax.axis_index("c")
    base_row = wid * (ROWS_PER_WORKER * NUM_FEATURES // 128)

    pltpu.sync_copy(foff_hbm, foff_v)
    pltpu.sync_copy(tmpl_hbm, tmpl_v)

    lane = lax.iota(jnp.int32, 16)
    head_mask = lane < 4
    cidx0 = jnp.maximum(lane - 4, 0)
    cidx1 = lane + 12

    for ch in range(NUM_CHUNKS):
        roff = base_row + ch * IDX_ROWS
        pltpu.sync_copy(codes_hbm.at[pl.ds(roff, IDX_ROWS)], codes_v)

        def idx_body(q, _):
            for k in range(8):
                s = k * 16
                c = codes_v[q, pl.ds(s, 16)]
                idx_v[q, pl.ds(s, 16)] = c.astype(jnp.int32) + foff_v[q, pl.ds(s, 16)]
            return 0

        lax.fori_loop(0, IDX_ROWS, idx_body, 0)

        gathers = [
            pltpu.async_copy(
                feat_hbm.at[idx_v.at[j]],
                rows_v.at[pl.ds(j * GATHER_SLICE, GATHER_SLICE)],
                sem,
            )
            for j in range(NUM_GATHERS)
        ]
        for g in gathers:
            g.wait()

        def repack_body(q, _):
            for k in range(4):
                r = q * 4 + k
                f = lax.rem(r, NUM_FEATURES)
                t = tmpl_v[f, :]
                rsplat = jnp.full((16,), r, jnp.int32)
                g0 = plsc.load_gather(rows_v, [rsplat, cidx0])
                ve = jnp.where(head_mask, t, g0)
                vo = plsc.load_gather(rows_v, [rsplat, cidx1])
                out_tile[q, pl.ds(k * 32, 16)] = ve
                out_tile[q, pl.ds(k * 32 + 16, 16)] = vo
            return 0

        lax.fori_loop(0, CFLAT // 4, repack_body, 0)

        pltpu.sync_copy(out_tile, out_hbm.at[pl.ds(roff * 32, CFLAT // 4)])


@jax.jit
def _run(feature_embedding, codes_2d, tmpl, foff):
    mesh = plsc.VectorSubcoreMesh(core_axis_name="c", subcore_axis_name="s")
    k = functools.partial(
        pl.kernel,
        mesh=mesh,
        out_type=jax.ShapeDtypeStruct((BATCH * NUM_FEATURES * OUT_DIM // 128, 128), jnp.float32),
        scratch_types=[
            pltpu.VMEM((IDX_ROWS, 128), jnp.float32),
            pltpu.VMEM((IDX_ROWS, 128), jnp.int32),
            pltpu.VMEM((IDX_ROWS, 128), jnp.int32),
            pltpu.VMEM((CFLAT, OUT_DIM), jnp.float32),
            pltpu.VMEM((CFLAT // 4, 128), jnp.float32),
            pltpu.VMEM((NUM_FEATURES, 16), jnp.float32),
            pltpu.SemaphoreType.DMA,
        ],
        compiler_params=pltpu.CompilerParams(use_tc_tiling_on_sc=False,
                                             needs_layout_passes=False),
    )(_sc_body)
    return k(feature_embedding, codes_2d, tmpl, foff)


def kernel(inputs, feature_embedding, shared_embedding):
    feature_embedding = jnp.pad(feature_embedding, ((0, 0), (0, OUT_DIM - FEATURE_DIM)))
    codes_2d = inputs.reshape(BATCH * NUM_FEATURES // 128, 128)
    tmpl = jnp.pad(shared_embedding, ((0, 0), (0, 12)))
    out = _run(feature_embedding, codes_2d, tmpl, jnp.asarray(_FOFF))
    return out.reshape(BATCH, NUM_FEATURES, OUT_DIM)
